# Initial kernel scaffold; baseline (speedup 1.0000x reference)
import functools

import jax
import jax.numpy as jnp
from jax import lax
from jax.experimental import pallas as pl
from jax.experimental.pallas import tpu as pltpu
from jax.experimental.pallas import tpu_sc as plsc

B, T, N, E, H, C = 2, 12, 10000, 160000, 32, 3
G = 2 * B
K = 125
NCH = (2 * E) // 16 // K
ROWS_PER_TILE = (2 * N) // 16


def _sc_agg_body(h_hbm, src_hbm, dst_hbm, ew_hbm, out_hbm,
                 srcv, dstv, eww, rows, acc, sem):
    c = lax.axis_index("c")
    s = lax.axis_index("s")
    pltpu.sync_copy(src_hbm.at[c, s], srcv)
    pltpu.sync_copy(dst_hbm.at[c, s], dstv)
    pltpu.sync_copy(ew_hbm.at[c, s], eww)

    def _zrow(i, _):
        rows[i, pl.ds(0, 16)] = jnp.zeros((16,), jnp.float32)
        rows[i, pl.ds(16, 16)] = jnp.zeros((16,), jnp.float32)
        return 0
    lax.fori_loop(0, K, _zrow, 0)

    def _zacc(k, _):
        pltpu.sync_copy(rows, acc.at[pl.ds(s * ROWS_PER_TILE + k * K, K)])
        return 0
    lax.fori_loop(0, ROWS_PER_TILE // K, _zacc, 0)
    plsc.subcore_barrier()

    def _chunk(j, _):
        pltpu.async_copy(h_hbm.at[srcv.at[j]], rows, sem).wait()

        def _mul(i5, _):
            for u in range(5):
                i = i5 * 5 + u
                w = eww[j, i]
                rows[i, pl.ds(0, 16)] = rows[i, pl.ds(0, 16)] * w
                rows[i, pl.ds(16, 16)] = rows[i, pl.ds(16, 16)] * w
            return 0
        lax.fori_loop(0, K // 5, _mul, 0)

        pltpu.sync_copy(rows, acc.at[dstv.at[j]], add=True)
        return 0
    lax.fori_loop(0, NCH, _chunk, 0)
    plsc.subcore_barrier()

    pltpu.sync_copy(acc.at[pl.ds(s * ROWS_PER_TILE, ROWS_PER_TILE)],
                    out_hbm.at[pl.ds(c * 2 * N + s * ROWS_PER_TILE, ROWS_PER_TILE)])


_sc_agg = pl.kernel(
    _sc_agg_body,
    out_type=jax.ShapeDtypeStruct((G * N, H), jnp.float32),
    mesh=plsc.VectorSubcoreMesh(core_axis_name="c", subcore_axis_name="s"),
    scratch_types=[
        pltpu.VMEM((NCH, K), jnp.int32),
        pltpu.VMEM((NCH, K), jnp.int32),
        pltpu.VMEM((NCH, K), jnp.float32),
        pltpu.VMEM((K, H), jnp.float32),
        pltpu.VMEM_SHARED((2 * N, H), jnp.float32),
        pltpu.SemaphoreType.DMA,
    ],
)


def _xwb_body(inp_ref, wi_ref, b_ref, out_ref):
    x = inp_ref[0, 0]
    wi = wi_ref[0]
    bb = b_ref[0]
    out_ref[0, 0] = jnp.dot(x, wi, preferred_element_type=jnp.float32) + bb


def _step0_body(xwb_ref, out_ref):
    out_ref[0] = jnp.tanh(xwb_ref[0])


def _step_body(agg_ref, xwb_ref, wh_ref, out_ref):
    out_ref[0] = jnp.tanh(
        xwb_ref[0]
        + jnp.dot(agg_ref[0], wh_ref[0], preferred_element_type=jnp.float32))


def _dec_body(hs_f_ref, hs_b_ref, xm_ref, mask_ref, w1_ref, b1_ref,
              w2_ref, b2_ref, res_ref, imp_ref):
    hc = jnp.concatenate([hs_f_ref[0, 0], hs_b_ref[0, 0]], axis=-1)
    m = jax.nn.relu(jnp.dot(hc, w1_ref[...],
                            preferred_element_type=jnp.float32) + b1_ref[...])
    imp = jnp.sum(m * w2_ref[...], axis=-1) + b2_ref[0, 0]
    xm = xm_ref[0, 0]
    mk = mask_ref[0, 0]
    imp_ref[0, 0] = imp
    res_ref[0, 0] = mk * xm + (1.0 - mk) * imp


def kernel(x, input_mask, time_gap_matrix, edge_index, edge_weights,
           Wi_f, Wh_f, b_f, Wi_b, Wh_b, b_b, W1, b1, W2, b2):
    src = edge_index[0]
    dst = edge_index[1]
    noise = jax.random.uniform(jax.random.key(42), x.shape, dtype=x.dtype) * 0.01
    xm = input_mask * x + (1.0 - input_mask) * noise

    inp = jnp.stack([xm, input_mask, time_gap_matrix], axis=-1)
    inp_g = jnp.concatenate([inp, jnp.flip(inp, axis=1)], axis=0)
    Wi_g = jnp.stack([Wi_f, Wi_f, Wi_b, Wi_b], axis=0)
    b_g = jnp.stack([b_f, b_f, b_b, b_b], axis=0)[:, None, :]
    Wh_g = jnp.stack([Wh_f, Wh_f, Wh_b, Wh_b], axis=0)

    gsrc = jnp.concatenate([src, src + N, src + 2 * N, src + 3 * N])
    gdst = jnp.concatenate([dst, dst + N, dst, dst + N])
    gew = jnp.concatenate([edge_weights] * G)
    gsrc = gsrc.reshape(2, 16, NCH, K)
    gdst = gdst.reshape(2, 16, NCH, K)
    gew = gew.reshape(2, 16, NCH, K)

    xwb = pl.pallas_call(
        _xwb_body,
        grid=(G, T),
        in_specs=[
            pl.BlockSpec((1, 1, N, C), lambda g, t: (g, t, 0, 0)),
            pl.BlockSpec((1, C, H), lambda g, t: (g, 0, 0)),
            pl.BlockSpec((1, 1, H), lambda g, t: (g, 0, 0)),
        ],
        out_specs=pl.BlockSpec((1, 1, N, H), lambda g, t: (g, t, 0, 0)),
        out_shape=jax.ShapeDtypeStruct((G, T, N, H), jnp.float32),
    )(inp_g, Wi_g, b_g)

    step0 = pl.pallas_call(
        _step0_body,
        grid=(G,),
        in_specs=[pl.BlockSpec((1, N, H), lambda g: (g, 0, 0))],
        out_specs=pl.BlockSpec((1, N, H), lambda g: (g, 0, 0)),
        out_shape=jax.ShapeDtypeStruct((G, N, H), jnp.float32),
    )

    step = pl.pallas_call(
        _step_body,
        grid=(G,),
        in_specs=[
            pl.BlockSpec((1, N, H), lambda g: (g, 0, 0)),
            pl.BlockSpec((1, N, H), lambda g: (g, 0, 0)),
            pl.BlockSpec((1, H, H), lambda g: (g, 0, 0)),
        ],
        out_specs=pl.BlockSpec((1, N, H), lambda g: (g, 0, 0)),
        out_shape=jax.ShapeDtypeStruct((G, N, H), jnp.float32),
    )

    h = step0(xwb[:, 0])
    h_list = [h]
    for t in range(1, T):
        agg = _sc_agg(h.reshape(G * N, H), gsrc, gdst, gew)
        h = step(agg.reshape(G, N, H), xwb[:, t], Wh_g)
        h_list.append(h)
    hs = jnp.stack(h_list, axis=1)

    res, imp = pl.pallas_call(
        _dec_body,
        grid=(B, T),
        in_specs=[
            pl.BlockSpec((1, 1, N, H), lambda b, t: (b, t, 0, 0)),
            pl.BlockSpec((1, 1, N, H), lambda b, t: (B + b, T - 1 - t, 0, 0)),
            pl.BlockSpec((1, 1, N), lambda b, t: (b, t, 0)),
            pl.BlockSpec((1, 1, N), lambda b, t: (b, t, 0)),
            pl.BlockSpec((2 * H, H), lambda b, t: (0, 0)),
            pl.BlockSpec((1, H), lambda b, t: (0, 0)),
            pl.BlockSpec((1, H), lambda b, t: (0, 0)),
            pl.BlockSpec((1, 1), lambda b, t: (0, 0)),
        ],
        out_specs=[
            pl.BlockSpec((1, 1, N), lambda b, t: (b, t, 0)),
            pl.BlockSpec((1, 1, N), lambda b, t: (b, t, 0)),
        ],
        out_shape=[
            jax.ShapeDtypeStruct((B, T, N), jnp.float32),
            jax.ShapeDtypeStruct((B, T, N), jnp.float32),
        ],
    )(hs, hs, xm, input_mask, W1, b1[None, :], W2.reshape(1, H),
      b2.reshape(1, 1))
    return (res, imp)


# speedup vs baseline: 89.9862x; 89.9862x over previous
import functools

import jax
import jax.numpy as jnp
from jax import lax
from jax.experimental import pallas as pl
from jax.experimental.pallas import tpu as pltpu
from jax.experimental.pallas import tpu_sc as plsc

B, T, N, E, H, C = 2, 12, 10000, 160000, 32, 3
G = 2 * B
K = 128
NCH = 160
PAD = 16 * NCH * K - 2 * E
SLAB = 1248
ZCH = 128
NZ = 10


def _sc_agg_body(h_hbm, src_hbm, dst_hbm, ew_hbm, out_hbm,
                 srcv, dstv, eww, rows, zrow, acc, sem):
    c = lax.axis_index("c")
    s = lax.axis_index("s")
    pltpu.sync_copy(src_hbm.at[c, s], srcv)
    pltpu.sync_copy(dst_hbm.at[c, s], dstv)
    pltpu.sync_copy(ew_hbm.at[c, s], eww)

    def _zr(i, _):
        zrow[i, pl.ds(0, 16)] = jnp.zeros((16,), jnp.float32)
        zrow[i, pl.ds(16, 16)] = jnp.zeros((16,), jnp.float32)
        return 0
    lax.fori_loop(0, ZCH, _zr, 0)

    base = s * SLAB
    def _zacc(k, _):
        pltpu.sync_copy(zrow, acc.at[pl.ds(base + k * ZCH, ZCH)])
        return 0
    lax.fori_loop(0, NZ, _zacc, 0)
    plsc.subcore_barrier()

    def _chunk(j, _):
        pltpu.async_copy(h_hbm.at[srcv.at[j]], rows, sem).wait()

        def _blk(q, _):
            wv = eww[j, pl.ds(q * 16, 16)]
            for i in range(16):
                ws = jnp.full((16,), wv[i], jnp.float32)
                r = q * 16 + i
                rows[r, pl.ds(0, 16)] = rows[r, pl.ds(0, 16)] * ws
                rows[r, pl.ds(16, 16)] = rows[r, pl.ds(16, 16)] * ws
            return 0
        lax.fori_loop(0, K // 16, _blk, 0)

        pltpu.sync_copy(rows, acc.at[dstv.at[j]], add=True)
        return 0
    lax.fori_loop(0, NCH, _chunk, 0)
    plsc.subcore_barrier()

    def _out(k, _):
        pltpu.sync_copy(acc.at[pl.ds(base + k * ZCH, ZCH)],
                        out_hbm.at[pl.ds(c * 2 * N + base + k * ZCH, ZCH)])
        return 0
    lax.fori_loop(0, NZ, _out, 0)


_sc_agg = pl.kernel(
    _sc_agg_body,
    out_type=jax.ShapeDtypeStruct((G * N, H), jnp.float32),
    mesh=plsc.VectorSubcoreMesh(core_axis_name="c", subcore_axis_name="s"),
    compiler_params=pltpu.CompilerParams(use_tc_tiling_on_sc=False),
    scratch_types=[
        pltpu.VMEM((NCH, K), jnp.int32),
        pltpu.VMEM((NCH, K), jnp.int32),
        pltpu.VMEM((NCH, K), jnp.float32),
        pltpu.VMEM((K, H), jnp.float32),
        pltpu.VMEM((ZCH, H), jnp.float32),
        pltpu.VMEM_SHARED((2 * N, H), jnp.float32),
        pltpu.SemaphoreType.DMA,
    ],
)


def _xwb_body(inp_ref, wi_ref, b_ref, out_ref):
    x = inp_ref[0, 0]
    wi = wi_ref[0]
    bb = b_ref[0]
    out_ref[0, 0] = jnp.dot(x, wi, preferred_element_type=jnp.float32) + bb


def _step0_body(xwb_ref, out_ref):
    out_ref[0] = jnp.tanh(xwb_ref[0])


def _step_body(agg_ref, xwb_ref, wh_ref, out_ref):
    out_ref[0] = jnp.tanh(
        xwb_ref[0]
        + jnp.dot(agg_ref[0], wh_ref[0], preferred_element_type=jnp.float32))


def _dec_body(hs_f_ref, hs_b_ref, xm_ref, mask_ref, w1_ref, b1_ref,
              w2_ref, b2_ref, res_ref, imp_ref):
    hc = jnp.concatenate([hs_f_ref[0, 0], hs_b_ref[0, 0]], axis=-1)
    m = jax.nn.relu(jnp.dot(hc, w1_ref[...],
                            preferred_element_type=jnp.float32) + b1_ref[...])
    imp = jnp.sum(m * w2_ref[...], axis=-1) + b2_ref[0, 0]
    xm = xm_ref[0, 0, 0]
    mk = mask_ref[0, 0, 0]
    imp_ref[0, 0, 0] = imp
    res_ref[0, 0, 0] = mk * xm + (1.0 - mk) * imp


def kernel(x, input_mask, time_gap_matrix, edge_index, edge_weights,
           Wi_f, Wh_f, b_f, Wi_b, Wh_b, b_b, W1, b1, W2, b2):
    src = edge_index[0]
    dst = edge_index[1]
    noise = jax.random.uniform(jax.random.key(42), x.shape, dtype=x.dtype) * 0.01
    xm = input_mask * x + (1.0 - input_mask) * noise

    inp = jnp.stack([xm, input_mask, time_gap_matrix], axis=-1)
    inp_g = jnp.concatenate([inp, jnp.flip(inp, axis=1)], axis=0)
    Wi_g = jnp.stack([Wi_f, Wi_f, Wi_b, Wi_b], axis=0)
    b_g = jnp.stack([b_f, b_f, b_b, b_b], axis=0)[:, None, :]
    Wh_g = jnp.stack([Wh_f, Wh_f, Wh_b, Wh_b], axis=0)

    pad_i = jnp.zeros((PAD,), jnp.int32)
    pad_f = jnp.zeros((PAD,), jnp.float32)
    gsrc = jnp.stack([
        jnp.concatenate([src, src + N, pad_i]),
        jnp.concatenate([src + 2 * N, src + 3 * N, pad_i]),
    ]).reshape(2, 16, NCH, K)
    gdst = jnp.stack([
        jnp.concatenate([dst, dst + N, pad_i]),
        jnp.concatenate([dst, dst + N, pad_i]),
    ]).reshape(2, 16, NCH, K)
    gew = jnp.stack([
        jnp.concatenate([edge_weights, edge_weights, pad_f]),
        jnp.concatenate([edge_weights, edge_weights, pad_f]),
    ]).reshape(2, 16, NCH, K)

    xwb = pl.pallas_call(
        _xwb_body,
        grid=(G, T),
        in_specs=[
            pl.BlockSpec((1, 1, N, C), lambda g, t: (g, t, 0, 0)),
            pl.BlockSpec((1, C, H), lambda g, t: (g, 0, 0)),
            pl.BlockSpec((1, 1, H), lambda g, t: (g, 0, 0)),
        ],
        out_specs=pl.BlockSpec((1, 1, N, H), lambda g, t: (g, t, 0, 0)),
        out_shape=jax.ShapeDtypeStruct((G, T, N, H), jnp.float32),
    )(inp_g, Wi_g, b_g)

    step0 = pl.pallas_call(
        _step0_body,
        grid=(G,),
        in_specs=[pl.BlockSpec((1, N, H), lambda g: (g, 0, 0))],
        out_specs=pl.BlockSpec((1, N, H), lambda g: (g, 0, 0)),
        out_shape=jax.ShapeDtypeStruct((G, N, H), jnp.float32),
    )

    step = pl.pallas_call(
        _step_body,
        grid=(G,),
        in_specs=[
            pl.BlockSpec((1, N, H), lambda g: (g, 0, 0)),
            pl.BlockSpec((1, N, H), lambda g: (g, 0, 0)),
            pl.BlockSpec((1, H, H), lambda g: (g, 0, 0)),
        ],
        out_specs=pl.BlockSpec((1, N, H), lambda g: (g, 0, 0)),
        out_shape=jax.ShapeDtypeStruct((G, N, H), jnp.float32),
    )

    h = step0(xwb[:, 0])
    h_list = [h]
    for t in range(1, T):
        agg = _sc_agg(h.reshape(G * N, H), gsrc, gdst, gew)
        h = step(agg.reshape(G, N, H), xwb[:, t], Wh_g)
        h_list.append(h)
    hs = jnp.stack(h_list, axis=1)

    res, imp = pl.pallas_call(
        _dec_body,
        grid=(B, T),
        in_specs=[
            pl.BlockSpec((1, 1, N, H), lambda b, t: (b, t, 0, 0)),
            pl.BlockSpec((1, 1, N, H), lambda b, t: (B + b, t, 0, 0)),
            pl.BlockSpec((1, 1, 1, N), lambda b, t: (b, t, 0, 0)),
            pl.BlockSpec((1, 1, 1, N), lambda b, t: (b, t, 0, 0)),
            pl.BlockSpec((2 * H, H), lambda b, t: (0, 0)),
            pl.BlockSpec((1, H), lambda b, t: (0, 0)),
            pl.BlockSpec((1, H), lambda b, t: (0, 0)),
            pl.BlockSpec((1, 1), lambda b, t: (0, 0)),
        ],
        out_specs=[
            pl.BlockSpec((1, 1, 1, N), lambda b, t: (b, t, 0, 0)),
            pl.BlockSpec((1, 1, 1, N), lambda b, t: (b, t, 0, 0)),
        ],
        out_shape=[
            jax.ShapeDtypeStruct((B, T, 1, N), jnp.float32),
            jax.ShapeDtypeStruct((B, T, 1, N), jnp.float32),
        ],
    )(hs, hs, xm.reshape(B, T, 1, N), input_mask.reshape(B, T, 1, N),
      W1, b1[None, :], W2.reshape(1, H), b2.reshape(1, 1))
    return (res.reshape(B, T, N), imp.reshape(B, T, N))


# speedup vs baseline: 110.5931x; 1.2290x over previous
import functools

import jax
import jax.numpy as jnp
from jax import lax
from jax.experimental import pallas as pl
from jax.experimental.pallas import tpu as pltpu
from jax.experimental.pallas import tpu_sc as plsc

B, T, N, E, H, C = 2, 12, 10000, 160000, 32, 3
G = 2 * B
K = 128
NCH = 160
PAD = 16 * NCH * K - 2 * E
SLAB = 1248
ZCH = 128
NZ = 10


NBUF = 4


def _sc_agg_body(h_hbm, src_hbm, dst_hbm, ew_hbm, out_hbm,
                 srcv, dstv, eww, rows, zrow, acc, sems, sem2):
    c = lax.axis_index("c")
    s = lax.axis_index("s")
    pltpu.async_copy(src_hbm.at[c, s], srcv, sem2)
    pltpu.async_copy(dst_hbm.at[c, s], dstv, sem2)
    pltpu.async_copy(ew_hbm.at[c, s], eww, sem2)

    def _zr(i, _):
        zrow[i, pl.ds(0, 16)] = jnp.zeros((16,), jnp.float32)
        zrow[i, pl.ds(16, 16)] = jnp.zeros((16,), jnp.float32)
        return 0
    lax.fori_loop(0, ZCH, _zr, 0)

    pltpu.make_async_copy(src_hbm.at[c, s], srcv, sem2).wait()
    pltpu.make_async_copy(dst_hbm.at[c, s], dstv, sem2).wait()
    pltpu.make_async_copy(ew_hbm.at[c, s], eww, sem2).wait()

    base = s * SLAB
    def _zacc(k, _):
        pltpu.async_copy(zrow, acc.at[pl.ds(base + k * ZCH, ZCH)], sem2)
        return 0
    lax.fori_loop(0, NZ, _zacc, 0)
    def _zw(k, _):
        pltpu.make_async_copy(zrow, acc.at[pl.ds(base + k * ZCH, ZCH)],
                              sem2).wait()
        return 0
    lax.fori_loop(0, NZ, _zw, 0)
    plsc.subcore_barrier()

    for b in range(NBUF):
        pltpu.async_copy(h_hbm.at[srcv.at[b]], rows.at[b], sems.at[b])

    def _outer(g, _):
        for b in range(NBUF):
            j = g * NBUF + b
            pltpu.make_async_copy(h_hbm.at[srcv.at[j]], rows.at[b],
                                  sems.at[b]).wait()

            def _blk(q, _):
                wv = eww[j, pl.ds(q * 16, 16)]
                for i in range(16):
                    ws = jnp.full((16,), wv[i], jnp.float32)
                    r = q * 16 + i
                    rows[b, r, pl.ds(0, 16)] = rows[b, r, pl.ds(0, 16)] * ws
                    rows[b, r, pl.ds(16, 16)] = rows[b, r, pl.ds(16, 16)] * ws
                return 0
            lax.fori_loop(0, K // 16, _blk, 0)

            pltpu.sync_copy(rows.at[b], acc.at[dstv.at[j]], add=True)

            jn = j + NBUF
            @pl.when(jn < NCH)
            def _():
                pltpu.async_copy(h_hbm.at[srcv.at[jn]], rows.at[b],
                                 sems.at[b])
        return 0
    lax.fori_loop(0, NCH // NBUF, _outer, 0)
    plsc.subcore_barrier()

    def _out(k, _):
        pltpu.async_copy(acc.at[pl.ds(base + k * ZCH, ZCH)],
                         out_hbm.at[pl.ds(c * 2 * N + base + k * ZCH, ZCH)],
                         sem2)
        return 0
    lax.fori_loop(0, NZ, _out, 0)
    def _outw(k, _):
        pltpu.make_async_copy(
            acc.at[pl.ds(base + k * ZCH, ZCH)],
            out_hbm.at[pl.ds(c * 2 * N + base + k * ZCH, ZCH)], sem2).wait()
        return 0
    lax.fori_loop(0, NZ, _outw, 0)


_sc_agg = pl.kernel(
    _sc_agg_body,
    out_type=jax.ShapeDtypeStruct((G * N, H), jnp.float32),
    mesh=plsc.VectorSubcoreMesh(core_axis_name="c", subcore_axis_name="s"),
    compiler_params=pltpu.CompilerParams(use_tc_tiling_on_sc=False),
    scratch_types=[
        pltpu.VMEM((NCH, K), jnp.int32),
        pltpu.VMEM((NCH, K), jnp.int32),
        pltpu.VMEM((NCH, K), jnp.float32),
        pltpu.VMEM((NBUF, K, H), jnp.float32),
        pltpu.VMEM((ZCH, H), jnp.float32),
        pltpu.VMEM_SHARED((2 * N, H), jnp.float32),
        pltpu.SemaphoreType.DMA((NBUF,)),
        pltpu.SemaphoreType.DMA,
    ],
)


def _xwb_body(inp_ref, wi_ref, b_ref, out_ref):
    x = inp_ref[0, 0]
    wi = wi_ref[0]
    bb = b_ref[0]
    out_ref[0, 0] = jnp.dot(x, wi, preferred_element_type=jnp.float32) + bb


def _step0_body(xwb_ref, out_ref):
    out_ref[0] = jnp.tanh(xwb_ref[0])


def _step_body(agg_ref, xwb_ref, wh_ref, out_ref):
    out_ref[0] = jnp.tanh(
        xwb_ref[0]
        + jnp.dot(agg_ref[0], wh_ref[0], preferred_element_type=jnp.float32))


def _dec_body(hs_f_ref, hs_b_ref, xm_ref, mask_ref, w1_ref, b1_ref,
              w2_ref, b2_ref, res_ref, imp_ref):
    hc = jnp.concatenate([hs_f_ref[0, 0], hs_b_ref[0, 0]], axis=-1)
    m = jax.nn.relu(jnp.dot(hc, w1_ref[...],
                            preferred_element_type=jnp.float32) + b1_ref[...])
    imp = jnp.sum(m * w2_ref[...], axis=-1) + b2_ref[0, 0]
    xm = xm_ref[0, 0, 0]
    mk = mask_ref[0, 0, 0]
    imp_ref[0, 0, 0] = imp
    res_ref[0, 0, 0] = mk * xm + (1.0 - mk) * imp


def kernel(x, input_mask, time_gap_matrix, edge_index, edge_weights,
           Wi_f, Wh_f, b_f, Wi_b, Wh_b, b_b, W1, b1, W2, b2):
    src = edge_index[0]
    dst = edge_index[1]
    noise = jax.random.uniform(jax.random.key(42), x.shape, dtype=x.dtype) * 0.01
    xm = input_mask * x + (1.0 - input_mask) * noise

    inp = jnp.stack([xm, input_mask, time_gap_matrix], axis=-1)
    inp_g = jnp.concatenate([inp, jnp.flip(inp, axis=1)], axis=0)
    Wi_g = jnp.stack([Wi_f, Wi_f, Wi_b, Wi_b], axis=0)
    b_g = jnp.stack([b_f, b_f, b_b, b_b], axis=0)[:, None, :]
    Wh_g = jnp.stack([Wh_f, Wh_f, Wh_b, Wh_b], axis=0)

    pad_i = jnp.zeros((PAD,), jnp.int32)
    pad_f = jnp.zeros((PAD,), jnp.float32)
    gsrc = jnp.stack([
        jnp.concatenate([src, src + N, pad_i]),
        jnp.concatenate([src + 2 * N, src + 3 * N, pad_i]),
    ]).reshape(2, 16, NCH, K)
    gdst = jnp.stack([
        jnp.concatenate([dst, dst + N, pad_i]),
        jnp.concatenate([dst, dst + N, pad_i]),
    ]).reshape(2, 16, NCH, K)
    gew = jnp.stack([
        jnp.concatenate([edge_weights, edge_weights, pad_f]),
        jnp.concatenate([edge_weights, edge_weights, pad_f]),
    ]).reshape(2, 16, NCH, K)

    xwb = pl.pallas_call(
        _xwb_body,
        grid=(G, T),
        in_specs=[
            pl.BlockSpec((1, 1, N, C), lambda g, t: (g, t, 0, 0)),
            pl.BlockSpec((1, C, H), lambda g, t: (g, 0, 0)),
            pl.BlockSpec((1, 1, H), lambda g, t: (g, 0, 0)),
        ],
        out_specs=pl.BlockSpec((1, 1, N, H), lambda g, t: (g, t, 0, 0)),
        out_shape=jax.ShapeDtypeStruct((G, T, N, H), jnp.float32),
    )(inp_g, Wi_g, b_g)

    step0 = pl.pallas_call(
        _step0_body,
        grid=(G,),
        in_specs=[pl.BlockSpec((1, N, H), lambda g: (g, 0, 0))],
        out_specs=pl.BlockSpec((1, N, H), lambda g: (g, 0, 0)),
        out_shape=jax.ShapeDtypeStruct((G, N, H), jnp.float32),
    )

    step = pl.pallas_call(
        _step_body,
        grid=(G,),
        in_specs=[
            pl.BlockSpec((1, N, H), lambda g: (g, 0, 0)),
            pl.BlockSpec((1, N, H), lambda g: (g, 0, 0)),
            pl.BlockSpec((1, H, H), lambda g: (g, 0, 0)),
        ],
        out_specs=pl.BlockSpec((1, N, H), lambda g: (g, 0, 0)),
        out_shape=jax.ShapeDtypeStruct((G, N, H), jnp.float32),
    )

    h = step0(xwb[:, 0])
    h_list = [h]
    for t in range(1, T):
        agg = _sc_agg(h.reshape(G * N, H), gsrc, gdst, gew)
        h = step(agg.reshape(G, N, H), xwb[:, t], Wh_g)
        h_list.append(h)
    hs = jnp.stack(h_list, axis=1)

    res, imp = pl.pallas_call(
        _dec_body,
        grid=(B, T),
        in_specs=[
            pl.BlockSpec((1, 1, N, H), lambda b, t: (b, t, 0, 0)),
            pl.BlockSpec((1, 1, N, H), lambda b, t: (B + b, t, 0, 0)),
            pl.BlockSpec((1, 1, 1, N), lambda b, t: (b, t, 0, 0)),
            pl.BlockSpec((1, 1, 1, N), lambda b, t: (b, t, 0, 0)),
            pl.BlockSpec((2 * H, H), lambda b, t: (0, 0)),
            pl.BlockSpec((1, H), lambda b, t: (0, 0)),
            pl.BlockSpec((1, H), lambda b, t: (0, 0)),
            pl.BlockSpec((1, 1), lambda b, t: (0, 0)),
        ],
        out_specs=[
            pl.BlockSpec((1, 1, 1, N), lambda b, t: (b, t, 0, 0)),
            pl.BlockSpec((1, 1, 1, N), lambda b, t: (b, t, 0, 0)),
        ],
        out_shape=[
            jax.ShapeDtypeStruct((B, T, 1, N), jnp.float32),
            jax.ShapeDtypeStruct((B, T, 1, N), jnp.float32),
        ],
    )(hs, hs, xm.reshape(B, T, 1, N), input_mask.reshape(B, T, 1, N),
      W1, b1[None, :], W2.reshape(1, H), b2.reshape(1, 1))
    return (res.reshape(B, T, N), imp.reshape(B, T, N))


# speedup vs baseline: 113.8601x; 1.0295x over previous
import functools

import jax
import jax.numpy as jnp
from jax import lax
from jax.experimental import pallas as pl
from jax.experimental.pallas import tpu as pltpu
from jax.experimental.pallas import tpu_sc as plsc

B, T, N, E, H, C = 2, 12, 10000, 160000, 32, 3
G = 2 * B
K = 128
NCH = 160
PAD = 16 * NCH * K - 2 * E
SLAB = 1248
ZCH = 128
NZ = 10


NBUF = 4
WBUF = 2


def _sc_agg_body(h_hbm, src_hbm, dst_hbm, ew_hbm, out_hbm,
                 srcv, dstv, eww, rows, wrows, zrow, acc, sems, ssems, sem2):
    c = lax.axis_index("c")
    s = lax.axis_index("s")
    pltpu.async_copy(src_hbm.at[c, s], srcv, sem2)
    pltpu.async_copy(dst_hbm.at[c, s], dstv, sem2)
    pltpu.async_copy(ew_hbm.at[c, s], eww, sem2)

    def _zr(i, _):
        zrow[i, pl.ds(0, 16)] = jnp.zeros((16,), jnp.float32)
        zrow[i, pl.ds(16, 16)] = jnp.zeros((16,), jnp.float32)
        return 0
    lax.fori_loop(0, ZCH, _zr, 0)

    pltpu.make_async_copy(src_hbm.at[c, s], srcv, sem2).wait()
    pltpu.make_async_copy(dst_hbm.at[c, s], dstv, sem2).wait()
    pltpu.make_async_copy(ew_hbm.at[c, s], eww, sem2).wait()

    base = s * SLAB
    def _zacc(k, _):
        pltpu.async_copy(zrow, acc.at[pl.ds(base + k * ZCH, ZCH)], sem2)
        return 0
    lax.fori_loop(0, NZ, _zacc, 0)
    def _zw(k, _):
        pltpu.make_async_copy(zrow, acc.at[pl.ds(base + k * ZCH, ZCH)],
                              sem2).wait()
        return 0
    lax.fori_loop(0, NZ, _zw, 0)
    plsc.subcore_barrier()

    for b in range(NBUF):
        pltpu.async_copy(h_hbm.at[srcv.at[b]], rows.at[b], sems.at[b])

    def _outer(g, _):
        for b in range(NBUF):
            j = g * NBUF + b

            wb = b % WBUF
            @pl.when(j >= WBUF)
            def _():
                pltpu.make_async_copy(
                    wrows.at[wb], acc.at[dstv.at[j - WBUF]],
                    ssems.at[wb]).wait()

            pltpu.make_async_copy(h_hbm.at[srcv.at[j]], rows.at[b],
                                  sems.at[b]).wait()

            def _blk(q, _):
                wv = eww[j, pl.ds(q * 16, 16)]
                for i in range(16):
                    ws = jnp.full((16,), wv[i], jnp.float32)
                    r = q * 16 + i
                    wrows[wb, r, pl.ds(0, 16)] = rows[b, r, pl.ds(0, 16)] * ws
                    wrows[wb, r, pl.ds(16, 16)] = rows[b, r, pl.ds(16, 16)] * ws
                return 0
            lax.fori_loop(0, K // 16, _blk, 0)

            pltpu.async_copy(wrows.at[wb], acc.at[dstv.at[j]], ssems.at[wb],
                             add=True)

            jn = j + NBUF
            @pl.when(jn < NCH)
            def _():
                pltpu.async_copy(h_hbm.at[srcv.at[jn]], rows.at[b],
                                 sems.at[b])
        return 0
    lax.fori_loop(0, NCH // NBUF, _outer, 0)

    for b in range(WBUF):
        jt = NCH - WBUF + b
        pltpu.make_async_copy(wrows.at[jt % WBUF], acc.at[dstv.at[jt]],
                              ssems.at[jt % WBUF]).wait()
    plsc.subcore_barrier()

    def _out(k, _):
        pltpu.async_copy(acc.at[pl.ds(base + k * ZCH, ZCH)],
                         out_hbm.at[pl.ds(c * 2 * N + base + k * ZCH, ZCH)],
                         sem2)
        return 0
    lax.fori_loop(0, NZ, _out, 0)
    def _outw(k, _):
        pltpu.make_async_copy(
            acc.at[pl.ds(base + k * ZCH, ZCH)],
            out_hbm.at[pl.ds(c * 2 * N + base + k * ZCH, ZCH)], sem2).wait()
        return 0
    lax.fori_loop(0, NZ, _outw, 0)


_sc_agg = pl.kernel(
    _sc_agg_body,
    out_type=jax.ShapeDtypeStruct((G * N, H), jnp.float32),
    mesh=plsc.VectorSubcoreMesh(core_axis_name="c", subcore_axis_name="s"),
    compiler_params=pltpu.CompilerParams(use_tc_tiling_on_sc=False),
    scratch_types=[
        pltpu.VMEM((NCH, K), jnp.int32),
        pltpu.VMEM((NCH, K), jnp.int32),
        pltpu.VMEM((NCH, K), jnp.float32),
        pltpu.VMEM((NBUF, K, H), jnp.float32),
        pltpu.VMEM((WBUF, K, H), jnp.float32),
        pltpu.VMEM((ZCH, H), jnp.float32),
        pltpu.VMEM_SHARED((2 * N, H), jnp.float32),
        pltpu.SemaphoreType.DMA((NBUF,)),
        pltpu.SemaphoreType.DMA((WBUF,)),
        pltpu.SemaphoreType.DMA,
    ],
)


def _xwb_body(inp_ref, wi_ref, b_ref, out_ref):
    x = inp_ref[0, 0]
    wi = wi_ref[0]
    bb = b_ref[0]
    out_ref[0, 0] = jnp.dot(x, wi, preferred_element_type=jnp.float32) + bb


def _step0_body(xwb_ref, out_ref):
    out_ref[0] = jnp.tanh(xwb_ref[0])


def _step_body(agg_ref, xwb_ref, wh_ref, out_ref):
    out_ref[0] = jnp.tanh(
        xwb_ref[0]
        + jnp.dot(agg_ref[0], wh_ref[0], preferred_element_type=jnp.float32))


def _dec_body(hs_f_ref, hs_b_ref, xm_ref, mask_ref, w1_ref, b1_ref,
              w2_ref, b2_ref, res_ref, imp_ref):
    hc = jnp.concatenate([hs_f_ref[0, 0], hs_b_ref[0, 0]], axis=-1)
    m = jax.nn.relu(jnp.dot(hc, w1_ref[...],
                            preferred_element_type=jnp.float32) + b1_ref[...])
    imp = jnp.sum(m * w2_ref[...], axis=-1) + b2_ref[0, 0]
    xm = xm_ref[0, 0, 0]
    mk = mask_ref[0, 0, 0]
    imp_ref[0, 0, 0] = imp
    res_ref[0, 0, 0] = mk * xm + (1.0 - mk) * imp


def kernel(x, input_mask, time_gap_matrix, edge_index, edge_weights,
           Wi_f, Wh_f, b_f, Wi_b, Wh_b, b_b, W1, b1, W2, b2):
    src = edge_index[0]
    dst = edge_index[1]
    noise = jax.random.uniform(jax.random.key(42), x.shape, dtype=x.dtype) * 0.01
    xm = input_mask * x + (1.0 - input_mask) * noise

    inp = jnp.stack([xm, input_mask, time_gap_matrix], axis=-1)
    inp_g = jnp.concatenate([inp, jnp.flip(inp, axis=1)], axis=0)
    Wi_g = jnp.stack([Wi_f, Wi_f, Wi_b, Wi_b], axis=0)
    b_g = jnp.stack([b_f, b_f, b_b, b_b], axis=0)[:, None, :]
    Wh_g = jnp.stack([Wh_f, Wh_f, Wh_b, Wh_b], axis=0)

    pad_i = jnp.zeros((PAD,), jnp.int32)
    pad_f = jnp.zeros((PAD,), jnp.float32)
    gsrc = jnp.stack([
        jnp.concatenate([src, src + N, pad_i]),
        jnp.concatenate([src + 2 * N, src + 3 * N, pad_i]),
    ]).reshape(2, 16, NCH, K)
    gdst = jnp.stack([
        jnp.concatenate([dst, dst + N, pad_i]),
        jnp.concatenate([dst, dst + N, pad_i]),
    ]).reshape(2, 16, NCH, K)
    gew = jnp.stack([
        jnp.concatenate([edge_weights, edge_weights, pad_f]),
        jnp.concatenate([edge_weights, edge_weights, pad_f]),
    ]).reshape(2, 16, NCH, K)

    xwb = pl.pallas_call(
        _xwb_body,
        grid=(G, T),
        in_specs=[
            pl.BlockSpec((1, 1, N, C), lambda g, t: (g, t, 0, 0)),
            pl.BlockSpec((1, C, H), lambda g, t: (g, 0, 0)),
            pl.BlockSpec((1, 1, H), lambda g, t: (g, 0, 0)),
        ],
        out_specs=pl.BlockSpec((1, 1, N, H), lambda g, t: (g, t, 0, 0)),
        out_shape=jax.ShapeDtypeStruct((G, T, N, H), jnp.float32),
    )(inp_g, Wi_g, b_g)

    step0 = pl.pallas_call(
        _step0_body,
        grid=(G,),
        in_specs=[pl.BlockSpec((1, N, H), lambda g: (g, 0, 0))],
        out_specs=pl.BlockSpec((1, N, H), lambda g: (g, 0, 0)),
        out_shape=jax.ShapeDtypeStruct((G, N, H), jnp.float32),
    )

    step = pl.pallas_call(
        _step_body,
        grid=(G,),
        in_specs=[
            pl.BlockSpec((1, N, H), lambda g: (g, 0, 0)),
            pl.BlockSpec((1, N, H), lambda g: (g, 0, 0)),
            pl.BlockSpec((1, H, H), lambda g: (g, 0, 0)),
        ],
        out_specs=pl.BlockSpec((1, N, H), lambda g: (g, 0, 0)),
        out_shape=jax.ShapeDtypeStruct((G, N, H), jnp.float32),
    )

    h = step0(xwb[:, 0])
    h_list = [h]
    for t in range(1, T):
        agg = _sc_agg(h.reshape(G * N, H), gsrc, gdst, gew)
        h = step(agg.reshape(G, N, H), xwb[:, t], Wh_g)
        h_list.append(h)
    hs = jnp.stack(h_list, axis=1)

    res, imp = pl.pallas_call(
        _dec_body,
        grid=(B, T),
        in_specs=[
            pl.BlockSpec((1, 1, N, H), lambda b, t: (b, t, 0, 0)),
            pl.BlockSpec((1, 1, N, H), lambda b, t: (B + b, t, 0, 0)),
            pl.BlockSpec((1, 1, 1, N), lambda b, t: (b, t, 0, 0)),
            pl.BlockSpec((1, 1, 1, N), lambda b, t: (b, t, 0, 0)),
            pl.BlockSpec((2 * H, H), lambda b, t: (0, 0)),
            pl.BlockSpec((1, H), lambda b, t: (0, 0)),
            pl.BlockSpec((1, H), lambda b, t: (0, 0)),
            pl.BlockSpec((1, 1), lambda b, t: (0, 0)),
        ],
        out_specs=[
            pl.BlockSpec((1, 1, 1, N), lambda b, t: (b, t, 0, 0)),
            pl.BlockSpec((1, 1, 1, N), lambda b, t: (b, t, 0, 0)),
        ],
        out_shape=[
            jax.ShapeDtypeStruct((B, T, 1, N), jnp.float32),
            jax.ShapeDtypeStruct((B, T, 1, N), jnp.float32),
        ],
    )(hs, hs, xm.reshape(B, T, 1, N), input_mask.reshape(B, T, 1, N),
      W1, b1[None, :], W2.reshape(1, H), b2.reshape(1, 1))
    return (res.reshape(B, T, N), imp.reshape(B, T, N))


# speedup vs baseline: 115.8724x; 1.0177x over previous
import functools

import jax
import jax.numpy as jnp
from jax import lax
from jax.experimental import pallas as pl
from jax.experimental.pallas import tpu as pltpu
from jax.experimental.pallas import tpu_sc as plsc

B, T, N, E, H, C = 2, 12, 10000, 160000, 32, 3
G = 2 * B
K = 128
NCH = 80
PAD = 16 * NCH * K - E
NPC = 10240
SLAB = 640
ZCH = 128
NZ = 5

NBUF = 8
WBUF = 4


def _sc_agg_body(h_hbm, src_hbm, dst_hbm, ew_hbm, out_hbm,
                 srcv, dstv, eww, rows, wrows, zrow, acc, sems, ssems, sem2):
    c = lax.axis_index("c")
    s = lax.axis_index("s")
    pltpu.async_copy(src_hbm.at[c, s], srcv, sem2)
    pltpu.async_copy(dst_hbm.at[c, s], dstv, sem2)
    pltpu.async_copy(ew_hbm.at[c, s], eww, sem2)

    def _zr(i, _):
        zrow[i, pl.ds(0, 16)] = jnp.zeros((16,), jnp.float32)
        zrow[i, pl.ds(16, 16)] = jnp.zeros((16,), jnp.float32)
        return 0
    lax.fori_loop(0, ZCH, _zr, 0)

    pltpu.make_async_copy(src_hbm.at[c, s], srcv, sem2).wait()
    pltpu.make_async_copy(dst_hbm.at[c, s], dstv, sem2).wait()
    pltpu.make_async_copy(ew_hbm.at[c, s], eww, sem2).wait()

    base = s * SLAB
    def _zacc(k, _):
        pltpu.async_copy(zrow, acc.at[pl.ds(base + k * ZCH, ZCH)], sem2)
        return 0
    lax.fori_loop(0, NZ, _zacc, 0)
    def _zw(k, _):
        pltpu.make_async_copy(zrow, acc.at[pl.ds(base + k * ZCH, ZCH)],
                              sem2).wait()
        return 0
    lax.fori_loop(0, NZ, _zw, 0)
    plsc.subcore_barrier()

    for b in range(NBUF):
        pltpu.async_copy(h_hbm.at[srcv.at[b]], rows.at[b], sems.at[b])

    def _outer(g, _):
        for b in range(NBUF):
            j = g * NBUF + b

            wb = b % WBUF
            @pl.when(j >= WBUF)
            def _():
                pltpu.make_async_copy(
                    wrows.at[wb], acc.at[dstv.at[j - WBUF]],
                    ssems.at[wb]).wait()

            pltpu.make_async_copy(h_hbm.at[srcv.at[j]], rows.at[b],
                                  sems.at[b]).wait()

            def _blk(q, _):
                wv = eww[j, pl.ds(q * 16, 16)]
                for i in range(16):
                    ws = jnp.full((16,), wv[i], jnp.float32)
                    r = q * 16 + i
                    wrows[wb, r, pl.ds(0, 16)] = rows[b, r, pl.ds(0, 16)] * ws
                    wrows[wb, r, pl.ds(16, 16)] = rows[b, r, pl.ds(16, 16)] * ws
                return 0
            lax.fori_loop(0, K // 16, _blk, 0)

            pltpu.async_copy(wrows.at[wb], acc.at[dstv.at[j]], ssems.at[wb],
                             add=True)

            jn = j + NBUF
            @pl.when(jn < NCH)
            def _():
                pltpu.async_copy(h_hbm.at[srcv.at[jn]], rows.at[b],
                                 sems.at[b])
        return 0
    lax.fori_loop(0, NCH // NBUF, _outer, 0)

    for b in range(WBUF):
        jt = NCH - WBUF + b
        pltpu.make_async_copy(wrows.at[jt % WBUF], acc.at[dstv.at[jt]],
                              ssems.at[jt % WBUF]).wait()
    plsc.subcore_barrier()

    def _out(k, _):
        pltpu.async_copy(acc.at[pl.ds(base + k * ZCH, ZCH)],
                         out_hbm.at[pl.ds(c * NPC + base + k * ZCH, ZCH)],
                         sem2)
        return 0
    lax.fori_loop(0, NZ, _out, 0)
    def _outw(k, _):
        pltpu.make_async_copy(
            acc.at[pl.ds(base + k * ZCH, ZCH)],
            out_hbm.at[pl.ds(c * NPC + base + k * ZCH, ZCH)], sem2).wait()
        return 0
    lax.fori_loop(0, NZ, _outw, 0)


_sc_agg = pl.kernel(
    _sc_agg_body,
    out_type=jax.ShapeDtypeStruct((2 * NPC, H), jnp.float32),
    mesh=plsc.VectorSubcoreMesh(core_axis_name="c", subcore_axis_name="s"),
    compiler_params=pltpu.CompilerParams(use_tc_tiling_on_sc=False),
    scratch_types=[
        pltpu.VMEM((NCH, K), jnp.int32),
        pltpu.VMEM((NCH, K), jnp.int32),
        pltpu.VMEM((NCH, K), jnp.float32),
        pltpu.VMEM((NBUF, K, H), jnp.float32),
        pltpu.VMEM((WBUF, K, H), jnp.float32),
        pltpu.VMEM((ZCH, H), jnp.float32),
        pltpu.VMEM_SHARED((NPC, H), jnp.float32),
        pltpu.SemaphoreType.DMA((NBUF,)),
        pltpu.SemaphoreType.DMA((WBUF,)),
        pltpu.SemaphoreType.DMA,
    ],
)


def _xwb_body(inp_ref, wi_ref, b_ref, out_ref):
    x = inp_ref[0, 0]
    wi = wi_ref[0]
    bb = b_ref[0]
    out_ref[0, 0] = jnp.dot(x, wi, preferred_element_type=jnp.float32) + bb


def _step0_body(xwb_ref, out_ref):
    out_ref[0] = jnp.tanh(xwb_ref[0])


def _step_body(agg_ref, xwb_ref, wh_ref, out_ref):
    out_ref[0] = jnp.tanh(
        xwb_ref[0]
        + jnp.dot(agg_ref[0], wh_ref[0], preferred_element_type=jnp.float32))


def _dec_body(hs_f_ref, hs_b_ref, xm_ref, mask_ref, w1_ref, b1_ref,
              w2_ref, b2_ref, res_ref, imp_ref):
    hc = jnp.concatenate([hs_f_ref[0, 0], hs_b_ref[0, 0]], axis=-1)
    m = jax.nn.relu(jnp.dot(hc, w1_ref[...],
                            preferred_element_type=jnp.float32) + b1_ref[...])
    imp = jnp.sum(m * w2_ref[...], axis=-1) + b2_ref[0, 0]
    xm = xm_ref[0, 0, 0]
    mk = mask_ref[0, 0, 0]
    imp_ref[0, 0, 0] = imp
    res_ref[0, 0, 0] = mk * xm + (1.0 - mk) * imp


def kernel(x, input_mask, time_gap_matrix, edge_index, edge_weights,
           Wi_f, Wh_f, b_f, Wi_b, Wh_b, b_b, W1, b1, W2, b2):
    src = edge_index[0]
    dst = edge_index[1]
    noise = jax.random.uniform(jax.random.key(42), x.shape, dtype=x.dtype) * 0.01
    xm = input_mask * x + (1.0 - input_mask) * noise

    inp = jnp.stack([xm, input_mask, time_gap_matrix], axis=-1)
    inp_g = jnp.concatenate([inp, jnp.flip(inp, axis=1)], axis=0)
    Wi_g = jnp.stack([Wi_f, Wi_f, Wi_b, Wi_b], axis=0)
    b_g = jnp.stack([b_f, b_f, b_b, b_b], axis=0)[:, None, :]

    pad_i = jnp.zeros((PAD,), jnp.int32)
    pad_f = jnp.zeros((PAD,), jnp.float32)
    gsrc = jnp.stack([
        jnp.concatenate([src, pad_i]),
        jnp.concatenate([src + N, pad_i]),
    ]).reshape(2, 16, NCH, K)
    gdst = jnp.stack([
        jnp.concatenate([dst, pad_i]),
        jnp.concatenate([dst, pad_i]),
    ]).reshape(2, 16, NCH, K)
    gew = jnp.stack([
        jnp.concatenate([edge_weights, pad_f]),
        jnp.concatenate([edge_weights, pad_f]),
    ]).reshape(2, 16, NCH, K)

    xwb = pl.pallas_call(
        _xwb_body,
        grid=(G, T),
        in_specs=[
            pl.BlockSpec((1, 1, N, C), lambda g, t: (g, t, 0, 0)),
            pl.BlockSpec((1, C, H), lambda g, t: (g, 0, 0)),
            pl.BlockSpec((1, 1, H), lambda g, t: (g, 0, 0)),
        ],
        out_specs=pl.BlockSpec((1, 1, N, H), lambda g, t: (g, t, 0, 0)),
        out_shape=jax.ShapeDtypeStruct((G, T, N, H), jnp.float32),
    )(inp_g, Wi_g, b_g)

    step0 = pl.pallas_call(
        _step0_body,
        grid=(G,),
        in_specs=[pl.BlockSpec((1, N, H), lambda g: (g, 0, 0))],
        out_specs=pl.BlockSpec((1, N, H), lambda g: (g, 0, 0)),
        out_shape=jax.ShapeDtypeStruct((G, N, H), jnp.float32),
    )

    step = pl.pallas_call(
        _step_body,
        grid=(2,),
        in_specs=[
            pl.BlockSpec((1, N, H), lambda g: (g, 0, 0)),
            pl.BlockSpec((1, N, H), lambda g: (g, 0, 0)),
            pl.BlockSpec((1, H, H), lambda g: (0, 0, 0)),
        ],
        out_specs=pl.BlockSpec((1, N, H), lambda g: (g, 0, 0)),
        out_shape=jax.ShapeDtypeStruct((2, N, H), jnp.float32),
    )

    h0 = step0(xwb[:, 0])
    ha, hb = h0[:2], h0[2:]
    Whf = Wh_f[None]
    Whb = Wh_b[None]
    ha_list, hb_list = [ha], [hb]
    for t in range(1, T):
        agg_a = _sc_agg(ha.reshape(2 * N, H), gsrc, gdst, gew)
        agg_b = _sc_agg(hb.reshape(2 * N, H), gsrc, gdst, gew)
        ha = step(agg_a.reshape(2, NPC, H), xwb[:2, t], Whf)
        hb = step(agg_b.reshape(2, NPC, H), xwb[2:, t], Whb)
        ha_list.append(ha)
        hb_list.append(hb)
    hs = jnp.concatenate([jnp.stack(ha_list, axis=1),
                          jnp.stack(hb_list, axis=1)], axis=0)

    res, imp = pl.pallas_call(
        _dec_body,
        grid=(B, T),
        in_specs=[
            pl.BlockSpec((1, 1, N, H), lambda b, t: (b, t, 0, 0)),
            pl.BlockSpec((1, 1, N, H), lambda b, t: (B + b, t, 0, 0)),
            pl.BlockSpec((1, 1, 1, N), lambda b, t: (b, t, 0, 0)),
            pl.BlockSpec((1, 1, 1, N), lambda b, t: (b, t, 0, 0)),
            pl.BlockSpec((2 * H, H), lambda b, t: (0, 0)),
            pl.BlockSpec((1, H), lambda b, t: (0, 0)),
            pl.BlockSpec((1, H), lambda b, t: (0, 0)),
            pl.BlockSpec((1, 1), lambda b, t: (0, 0)),
        ],
        out_specs=[
            pl.BlockSpec((1, 1, 1, N), lambda b, t: (b, t, 0, 0)),
            pl.BlockSpec((1, 1, 1, N), lambda b, t: (b, t, 0, 0)),
        ],
        out_shape=[
            jax.ShapeDtypeStruct((B, T, 1, N), jnp.float32),
            jax.ShapeDtypeStruct((B, T, 1, N), jnp.float32),
        ],
    )(hs, hs, xm.reshape(B, T, 1, N), input_mask.reshape(B, T, 1, N),
      W1, b1[None, :], W2.reshape(1, H), b2.reshape(1, 1))
    return (res.reshape(B, T, N), imp.reshape(B, T, N))


# speedup vs baseline: 115.9723x; 1.0009x over previous
import functools

import jax
import jax.numpy as jnp
from jax import lax
from jax.experimental import pallas as pl
from jax.experimental.pallas import tpu as pltpu
from jax.experimental.pallas import tpu_sc as plsc

B, T, N, E, H, C = 2, 12, 10000, 160000, 32, 3
G = 2 * B
K = 128
NCH = 80
PAD = 16 * NCH * K - E
NPC = 10240
SLAB = 640
ZCH = 128
NZ = 5

NBUF = 10
WBUF = 2


def _sc_agg_body(h_hbm, src_hbm, dst_hbm, ew_hbm, out_hbm,
                 srcv, dstv, eww, rows, wrows, zrow, acc, sems, ssems, sem2):
    c = lax.axis_index("c")
    s = lax.axis_index("s")
    pltpu.async_copy(src_hbm.at[c, s], srcv, sem2)
    pltpu.async_copy(dst_hbm.at[c, s], dstv, sem2)
    pltpu.async_copy(ew_hbm.at[c, s], eww, sem2)

    def _zr(i, _):
        zrow[i, pl.ds(0, 16)] = jnp.zeros((16,), jnp.float32)
        zrow[i, pl.ds(16, 16)] = jnp.zeros((16,), jnp.float32)
        return 0
    lax.fori_loop(0, ZCH, _zr, 0)

    pltpu.make_async_copy(src_hbm.at[c, s], srcv, sem2).wait()
    pltpu.make_async_copy(dst_hbm.at[c, s], dstv, sem2).wait()
    pltpu.make_async_copy(ew_hbm.at[c, s], eww, sem2).wait()

    base = s * SLAB
    def _zacc(k, _):
        pltpu.async_copy(zrow, acc.at[pl.ds(base + k * ZCH, ZCH)], sem2)
        return 0
    lax.fori_loop(0, NZ, _zacc, 0)
    def _zw(k, _):
        pltpu.make_async_copy(zrow, acc.at[pl.ds(base + k * ZCH, ZCH)],
                              sem2).wait()
        return 0
    lax.fori_loop(0, NZ, _zw, 0)
    plsc.subcore_barrier()

    for b in range(NBUF):
        pltpu.async_copy(h_hbm.at[srcv.at[b]], rows.at[b], sems.at[b])

    def _outer(g, _):
        for b in range(NBUF):
            j = g * NBUF + b

            wb = b % WBUF
            @pl.when(j >= WBUF)
            def _():
                pltpu.make_async_copy(
                    wrows.at[wb], acc.at[dstv.at[j - WBUF]],
                    ssems.at[wb]).wait()

            pltpu.make_async_copy(h_hbm.at[srcv.at[j]], rows.at[b],
                                  sems.at[b]).wait()

            def _blk(q, _):
                wv = eww[j, pl.ds(q * 16, 16)]
                for i in range(16):
                    ws = jnp.full((16,), wv[i], jnp.float32)
                    r = q * 16 + i
                    wrows[wb, r, pl.ds(0, 16)] = rows[b, r, pl.ds(0, 16)] * ws
                    wrows[wb, r, pl.ds(16, 16)] = rows[b, r, pl.ds(16, 16)] * ws
                return 0
            lax.fori_loop(0, K // 16, _blk, 0)

            pltpu.async_copy(wrows.at[wb], acc.at[dstv.at[j]], ssems.at[wb],
                             add=True)

            jn = j + NBUF
            @pl.when(jn < NCH)
            def _():
                pltpu.async_copy(h_hbm.at[srcv.at[jn]], rows.at[b],
                                 sems.at[b])
        return 0
    lax.fori_loop(0, NCH // NBUF, _outer, 0)

    for b in range(WBUF):
        jt = NCH - WBUF + b
        pltpu.make_async_copy(wrows.at[jt % WBUF], acc.at[dstv.at[jt]],
                              ssems.at[jt % WBUF]).wait()
    plsc.subcore_barrier()

    def _out(k, _):
        pltpu.async_copy(acc.at[pl.ds(base + k * ZCH, ZCH)],
                         out_hbm.at[pl.ds(c * NPC + base + k * ZCH, ZCH)],
                         sem2)
        return 0
    lax.fori_loop(0, NZ, _out, 0)
    def _outw(k, _):
        pltpu.make_async_copy(
            acc.at[pl.ds(base + k * ZCH, ZCH)],
            out_hbm.at[pl.ds(c * NPC + base + k * ZCH, ZCH)], sem2).wait()
        return 0
    lax.fori_loop(0, NZ, _outw, 0)


_sc_agg = pl.kernel(
    _sc_agg_body,
    out_type=jax.ShapeDtypeStruct((2 * NPC, H), jnp.float32),
    mesh=plsc.VectorSubcoreMesh(core_axis_name="c", subcore_axis_name="s"),
    compiler_params=pltpu.CompilerParams(use_tc_tiling_on_sc=False),
    scratch_types=[
        pltpu.VMEM((NCH, K), jnp.int32),
        pltpu.VMEM((NCH, K), jnp.int32),
        pltpu.VMEM((NCH, K), jnp.float32),
        pltpu.VMEM((NBUF, K, H), jnp.float32),
        pltpu.VMEM((WBUF, K, H), jnp.float32),
        pltpu.VMEM((ZCH, H), jnp.float32),
        pltpu.VMEM_SHARED((NPC, H), jnp.float32),
        pltpu.SemaphoreType.DMA((NBUF,)),
        pltpu.SemaphoreType.DMA((WBUF,)),
        pltpu.SemaphoreType.DMA,
    ],
)


def _xwb_body(inp_ref, wi_ref, b_ref, out_ref):
    x = inp_ref[0, 0]
    wi = wi_ref[0]
    bb = b_ref[0]
    out_ref[0, 0] = jnp.dot(x, wi, preferred_element_type=jnp.float32) + bb


def _step0_body(xwb_ref, out_ref):
    out_ref[0] = jnp.tanh(xwb_ref[0])


def _step_body(agg_ref, xwb_ref, wh_ref, out_ref):
    out_ref[0] = jnp.tanh(
        xwb_ref[0]
        + jnp.dot(agg_ref[0], wh_ref[0], preferred_element_type=jnp.float32))


def _dec_body(hs_f_ref, hs_b_ref, xm_ref, mask_ref, w1_ref, b1_ref,
              w2_ref, b2_ref, res_ref, imp_ref):
    hc = jnp.concatenate([hs_f_ref[0, 0], hs_b_ref[0, 0]], axis=-1)
    m = jax.nn.relu(jnp.dot(hc, w1_ref[...],
                            preferred_element_type=jnp.float32) + b1_ref[...])
    imp = jnp.sum(m * w2_ref[...], axis=-1) + b2_ref[0, 0]
    xm = xm_ref[0, 0, 0]
    mk = mask_ref[0, 0, 0]
    imp_ref[0, 0, 0] = imp
    res_ref[0, 0, 0] = mk * xm + (1.0 - mk) * imp


def kernel(x, input_mask, time_gap_matrix, edge_index, edge_weights,
           Wi_f, Wh_f, b_f, Wi_b, Wh_b, b_b, W1, b1, W2, b2):
    src = edge_index[0]
    dst = edge_index[1]
    noise = jax.random.uniform(jax.random.key(42), x.shape, dtype=x.dtype) * 0.01
    xm = input_mask * x + (1.0 - input_mask) * noise

    inp = jnp.stack([xm, input_mask, time_gap_matrix], axis=-1)
    inp_g = jnp.concatenate([inp, jnp.flip(inp, axis=1)], axis=0)
    Wi_g = jnp.stack([Wi_f, Wi_f, Wi_b, Wi_b], axis=0)
    b_g = jnp.stack([b_f, b_f, b_b, b_b], axis=0)[:, None, :]

    pad_i = jnp.zeros((PAD,), jnp.int32)
    pad_f = jnp.zeros((PAD,), jnp.float32)
    gsrc = jnp.stack([
        jnp.concatenate([src, pad_i]),
        jnp.concatenate([src + N, pad_i]),
    ]).reshape(2, 16, NCH, K)
    gdst = jnp.stack([
        jnp.concatenate([dst, pad_i]),
        jnp.concatenate([dst, pad_i]),
    ]).reshape(2, 16, NCH, K)
    gew = jnp.stack([
        jnp.concatenate([edge_weights, pad_f]),
        jnp.concatenate([edge_weights, pad_f]),
    ]).reshape(2, 16, NCH, K)

    xwb = pl.pallas_call(
        _xwb_body,
        grid=(G, T),
        in_specs=[
            pl.BlockSpec((1, 1, N, C), lambda g, t: (g, t, 0, 0)),
            pl.BlockSpec((1, C, H), lambda g, t: (g, 0, 0)),
            pl.BlockSpec((1, 1, H), lambda g, t: (g, 0, 0)),
        ],
        out_specs=pl.BlockSpec((1, 1, N, H), lambda g, t: (g, t, 0, 0)),
        out_shape=jax.ShapeDtypeStruct((G, T, N, H), jnp.float32),
    )(inp_g, Wi_g, b_g)

    step0 = pl.pallas_call(
        _step0_body,
        grid=(G,),
        in_specs=[pl.BlockSpec((1, N, H), lambda g: (g, 0, 0))],
        out_specs=pl.BlockSpec((1, N, H), lambda g: (g, 0, 0)),
        out_shape=jax.ShapeDtypeStruct((G, N, H), jnp.float32),
    )

    step = pl.pallas_call(
        _step_body,
        grid=(2,),
        in_specs=[
            pl.BlockSpec((1, N, H), lambda g: (g, 0, 0)),
            pl.BlockSpec((1, N, H), lambda g: (g, 0, 0)),
            pl.BlockSpec((1, H, H), lambda g: (0, 0, 0)),
        ],
        out_specs=pl.BlockSpec((1, N, H), lambda g: (g, 0, 0)),
        out_shape=jax.ShapeDtypeStruct((2, N, H), jnp.float32),
    )

    h0 = step0(xwb[:, 0])
    ha, hb = h0[:2], h0[2:]
    Whf = Wh_f[None]
    Whb = Wh_b[None]
    ha_list, hb_list = [ha], [hb]
    for t in range(1, T):
        agg_a = _sc_agg(ha.reshape(2 * N, H), gsrc, gdst, gew)
        agg_b = _sc_agg(hb.reshape(2 * N, H), gsrc, gdst, gew)
        ha = step(agg_a.reshape(2, NPC, H), xwb[:2, t], Whf)
        hb = step(agg_b.reshape(2, NPC, H), xwb[2:, t], Whb)
        ha_list.append(ha)
        hb_list.append(hb)
    hs = jnp.concatenate([jnp.stack(ha_list, axis=1),
                          jnp.stack(hb_list, axis=1)], axis=0)

    res, imp = pl.pallas_call(
        _dec_body,
        grid=(B, T),
        in_specs=[
            pl.BlockSpec((1, 1, N, H), lambda b, t: (b, t, 0, 0)),
            pl.BlockSpec((1, 1, N, H), lambda b, t: (B + b, t, 0, 0)),
            pl.BlockSpec((1, 1, 1, N), lambda b, t: (b, t, 0, 0)),
            pl.BlockSpec((1, 1, 1, N), lambda b, t: (b, t, 0, 0)),
            pl.BlockSpec((2 * H, H), lambda b, t: (0, 0)),
            pl.BlockSpec((1, H), lambda b, t: (0, 0)),
            pl.BlockSpec((1, H), lambda b, t: (0, 0)),
            pl.BlockSpec((1, 1), lambda b, t: (0, 0)),
        ],
        out_specs=[
            pl.BlockSpec((1, 1, 1, N), lambda b, t: (b, t, 0, 0)),
            pl.BlockSpec((1, 1, 1, N), lambda b, t: (b, t, 0, 0)),
        ],
        out_shape=[
            jax.ShapeDtypeStruct((B, T, 1, N), jnp.float32),
            jax.ShapeDtypeStruct((B, T, 1, N), jnp.float32),
        ],
    )(hs, hs, xm.reshape(B, T, 1, N), input_mask.reshape(B, T, 1, N),
      W1, b1[None, :], W2.reshape(1, H), b2.reshape(1, 1))
    return (res.reshape(B, T, N), imp.reshape(B, T, N))
